# baseline (speedup 1.0000x reference)
import functools

import jax
import jax.numpy as jnp
from jax import lax
from jax.experimental import pallas as pl
from jax.experimental.pallas import tpu as pltpu
from jax.experimental.pallas import tpu_sc as plsc

N = 10000
E = 320000
NPAD = 10240
NROWS = NPAD // 16
E_PER_TILE = E // 16
CHUNK = 128
NCHUNKS = 160
SUPER = 32
NGRP = SUPER // 8
EPT_PAD = NCHUNKS * CHUNK
DEG_W = 16


def _sc_mesh():
    return plsc.VectorSubcoreMesh(core_axis_name="c", subcore_axis_name="s")


@functools.partial(
    pl.kernel,
    mesh=_sc_mesh(),
    out_type=(
        jax.ShapeDtypeStruct((NPAD, DEG_W), jnp.float32),
        jax.ShapeDtypeStruct((NPAD, DEG_W), jnp.float32),
    ),
    scratch_types=[
        pltpu.VMEM((NCHUNKS, CHUNK), jnp.int32),
        pltpu.VMEM((CHUNK, DEG_W), jnp.float32),
        pltpu.VMEM_SHARED((NPAD, DEG_W), jnp.float32),
    ],
    compiler_params=pltpu.CompilerParams(use_tc_tiling_on_sc=False),
)
def _sc_degrees(src_h, dst_h, ones_h, zero_h, out_src, out_dst, idx_v, ones_v,
                acc):
    c = lax.axis_index("c")
    s = lax.axis_index("s")
    pltpu.sync_copy(zero_h, acc.at[pl.ds(s * NROWS, NROWS)])
    pltpu.sync_copy(ones_h, ones_v)
    plsc.subcore_barrier()

    def run(e_h, out_h):
        pltpu.sync_copy(e_h.at[s], idx_v)

        def body(i, carry):
            pltpu.sync_copy(ones_v, acc.at[idx_v.at[i]], add=True)
            return carry

        lax.fori_loop(0, NCHUNKS, body, 0)
        plsc.subcore_barrier()
        pltpu.sync_copy(acc.at[pl.ds(s * NROWS, NROWS)],
                        out_h.at[pl.ds(s * NROWS, NROWS)])

    @pl.when(c == 0)
    def _():
        run(src_h, out_src)

    @pl.when(c == 1)
    def _():
        run(dst_h, out_dst)


def _make_propagate(f_half, chunk, ring, nchunks, super_):
    nsuper = nchunks // super_
    lag = ring // 2
    ngrp = super_ // ring

    @functools.partial(
        pl.kernel,
        mesh=_sc_mesh(),
        out_type=(
            jax.ShapeDtypeStruct((NPAD, f_half), jnp.float32),
            jax.ShapeDtypeStruct((NPAD, f_half), jnp.float32),
        ),
        scratch_types=[
            pltpu.VMEM((super_, chunk), jnp.int32),
            pltpu.VMEM((super_, chunk), jnp.int32),
            [pltpu.VMEM((chunk, f_half), jnp.float32) for _ in range(ring)],
            [pltpu.SemaphoreType.DMA for _ in range(ring)],
            [pltpu.SemaphoreType.DMA for _ in range(ring)],
            pltpu.VMEM_SHARED((NPAD, f_half), jnp.float32),
        ],
        compiler_params=pltpu.CompilerParams(use_tc_tiling_on_sc=False),
    )
    def prop(ha, hb, srcA, dstA, srcB, dstB, zero_h, outa, outb,
             sidx, didx, rows, gsem, ssem, acc):
        c = lax.axis_index("c")
        s = lax.axis_index("s")
        pltpu.sync_copy(zero_h, acc.at[pl.ds(s * NROWS, NROWS)])
        plsc.subcore_barrier()

        def run(h_h, src_h, dst_h, out_h):
            def gather(i, b):
                pltpu.async_copy(h_h.at[sidx.at[i]], rows[b], gsem[b])

            def wait_gather(i, b):
                pltpu.make_async_copy(h_h.at[sidx.at[i]], rows[b],
                                      gsem[b]).wait()

            def scatter(i, b):
                pltpu.async_copy(rows[b], acc.at[didx.at[i]], ssem[b],
                                 add=True)

            def wait_scatter(i, b):
                pltpu.make_async_copy(rows[b], acc.at[didx.at[i]],
                                      ssem[b]).wait()

            def souter(t, carry):
                pltpu.sync_copy(src_h.at[s, pl.ds(t * super_, super_)], sidx)
                pltpu.sync_copy(dst_h.at[s, pl.ds(t * super_, super_)], didx)
                for b in range(lag):
                    gather(b, b)

                def group(m, carry2):
                    base = ring * m
                    for b in range(lag):
                        i = base + b

                        @pl.when(m > 0)
                        def _(i=i, b=b):
                            wait_scatter(i - lag, b + lag)

                        gather(i + lag, b + lag)
                        wait_gather(i, b)
                        scatter(i, b)
                    for b in range(lag, ring):
                        i = base + b

                        @pl.when(m < ngrp - 1)
                        def _(i=i, b=b):
                            wait_scatter(i - lag, b - lag)
                            gather(i + lag, b - lag)

                        wait_gather(i, b)
                        scatter(i, b)
                    return carry2

                lax.fori_loop(0, ngrp, group, 0)
                for b in range(ring):
                    wait_scatter(super_ - ring + b, b)
                return carry

            lax.fori_loop(0, nsuper, souter, 0)
            plsc.subcore_barrier()
            pltpu.sync_copy(acc.at[pl.ds(s * NROWS, NROWS)],
                            out_h.at[pl.ds(s * NROWS, NROWS)])

        @pl.when(c == 0)
        def _():
            run(ha, srcA, dstA, outa)

        @pl.when(c == 1)
        def _():
            run(hb, srcB, dstB, outb)

    return prop


_prop_l0 = _make_propagate(128, 64, 4, 160, 32)
_prop_l2 = _make_propagate(64, 128, 8, 80, 16)
_prop_l1 = _make_propagate(128, 64, 4, 320, 32)


RB = 1000
GRID = 10


def _row_spec(w):
    return pl.BlockSpec((RB, w), lambda i: (i, 0))


def _full_spec(shape):
    return pl.BlockSpec(shape, lambda i: tuple(0 for _ in shape))


def _tc_prep(x, od_h, id_h):
    def body(x_ref, od_ref, id_ref, u_ref, ns_ref, nd_ref):
        ns = lax.rsqrt(jnp.maximum(od_ref[...][:, :1], 1.0))
        nd = lax.rsqrt(jnp.maximum(id_ref[...][:, :1], 1.0))
        ns_ref[...] = ns
        nd_ref[...] = nd
        u_ref[...] = x_ref[...] * ns

    return pl.pallas_call(
        body,
        grid=(GRID,),
        in_specs=[_row_spec(128), _row_spec(DEG_W), _row_spec(DEG_W)],
        out_specs=[_row_spec(128), _row_spec(1), _row_spec(1)],
        out_shape=[
            jax.ShapeDtypeStruct((NPAD, 128), jnp.float32),
            jax.ShapeDtypeStruct((N, 1), jnp.float32),
            jax.ShapeDtypeStruct((N, 1), jnp.float32),
        ],
    )(x, od_h, id_h)


def _tc_layer0(ya, yb, ns, nd, W0, b0):
    def body(ya_ref, yb_ref, ns_ref, nd_ref, w_ref, b_ref, ua_ref, ub_ref):
        y = ya_ref[...] + yb_ref[...]
        z = jnp.dot(y, w_ref[...], preferred_element_type=jnp.float32)
        z = z * nd_ref[...] + b_ref[...]
        u = jnp.maximum(z, 0.0) * ns_ref[...]
        ua_ref[...] = u[:, :128]
        ub_ref[...] = u[:, 128:]

    return pl.pallas_call(
        body,
        grid=(GRID,),
        in_specs=[_row_spec(128), _row_spec(128), _row_spec(1), _row_spec(1),
                  _full_spec((128, 256)), _full_spec((1, 256))],
        out_specs=[_row_spec(128), _row_spec(128)],
        out_shape=[
            jax.ShapeDtypeStruct((NPAD, 128), jnp.float32),
            jax.ShapeDtypeStruct((NPAD, 128), jnp.float32),
        ],
    )(ya, yb, ns, nd, W0, b0)


def _tc_layer12(ya, yb, ns, nd, W1, b1, W2):
    def body(ya_ref, yb_ref, ns_ref, nd_ref, w1_ref, b1_ref, w2_ref, u_ref):
        z = jnp.dot(ya_ref[...], w1_ref[:128, :], preferred_element_type=jnp.float32)
        z += jnp.dot(yb_ref[...], w1_ref[128:, :], preferred_element_type=jnp.float32)
        z = z * nd_ref[...] + b1_ref[...]
        g = jnp.maximum(z, 0.0) * ns_ref[...]
        u_ref[...] = jnp.dot(g, w2_ref[...], preferred_element_type=jnp.float32)

    return pl.pallas_call(
        body,
        grid=(GRID,),
        in_specs=[_row_spec(128), _row_spec(128), _row_spec(1), _row_spec(1),
                  _full_spec((256, 256)), _full_spec((1, 256)),
                  _full_spec((256, 64))],
        out_specs=_row_spec(64),
        out_shape=jax.ShapeDtypeStruct((NPAD, 64), jnp.float32),
    )(ya, yb, ns, nd, W1, b1, W2)


def _tc_final(ya, yb, nd, b2):
    def body(ya_ref, yb_ref, nd_ref, b_ref, o_ref):
        z = ya_ref[...] + yb_ref[...]
        z = z * nd_ref[...] + b_ref[...]
        m = jnp.max(z, axis=1, keepdims=True)
        e = jnp.exp(z - m)
        lse = jnp.log(jnp.sum(e, axis=1, keepdims=True))
        o_ref[...] = (z - m) - lse

    return pl.pallas_call(
        body,
        grid=(GRID,),
        in_specs=[_row_spec(64), _row_spec(64), _row_spec(1),
                  _full_spec((1, 64))],
        out_specs=_row_spec(64),
        out_shape=jax.ShapeDtypeStruct((N, 64), jnp.float32),
    )(ya, yb, nd, b2)


def kernel(x, edge_index, W0, b0, W1, b1, W2, b2):
    pad = EPT_PAD - E_PER_TILE
    src3 = jnp.pad(edge_index[0].reshape(16, E_PER_TILE), ((0, 0), (0, pad)),
                   constant_values=N).reshape(16, NCHUNKS, CHUNK)
    dst3 = jnp.pad(edge_index[1].reshape(16, E_PER_TILE), ((0, 0), (0, pad)),
                   constant_values=N).reshape(16, NCHUNKS, CHUNK)
    ept2 = E // 32
    pad2 = 10240 - ept2
    srcE = jnp.pad(edge_index[0].reshape(2, 16, ept2),
                   ((0, 0), (0, 0), (0, pad2)), constant_values=N)
    dstE = jnp.pad(edge_index[1].reshape(2, 16, ept2),
                   ((0, 0), (0, 0), (0, pad2)), constant_values=N)

    ones_chunk = jnp.zeros((CHUNK, DEG_W), jnp.float32).at[:, 0].set(1.0)
    zero_deg = jnp.zeros((NROWS, DEG_W), jnp.float32)
    od_h, id_h = _sc_degrees(src3, dst3, ones_chunk, zero_deg)

    u0, ns, nd = _tc_prep(x, od_h, id_h)

    zero128 = jnp.zeros((NROWS, 128), jnp.float32)
    y0a, y0b = _prop_l0(u0, u0,
                        srcE[0].reshape(16, 160, 64),
                        dstE[0].reshape(16, 160, 64),
                        srcE[1].reshape(16, 160, 64),
                        dstE[1].reshape(16, 160, 64),
                        zero128)
    u1a, u1b = _tc_layer0(y0a, y0b, ns, nd, W0, b0.reshape(1, -1))

    src3b = src3.reshape(16, 320, 64)
    dst3b = dst3.reshape(16, 320, 64)
    y1a, y1b = _prop_l1(u1a, u1b, src3b, dst3b, src3b, dst3b, zero128)
    u2 = _tc_layer12(y1a, y1b, ns, nd, W1, b1.reshape(1, -1), W2)

    y2a, y2b = _prop_l2(u2, u2,
                        srcE[0].reshape(16, 80, 128),
                        dstE[0].reshape(16, 80, 128),
                        srcE[1].reshape(16, 80, 128),
                        dstE[1].reshape(16, 80, 128),
                        jnp.zeros((NROWS, 64), jnp.float32))
    return _tc_final(y2a, y2b, nd, b2.reshape(1, -1))


# speedup vs baseline: 1.1947x; 1.1947x over previous
import functools

import jax
import jax.numpy as jnp
from jax import lax
from jax.experimental import pallas as pl
from jax.experimental.pallas import tpu as pltpu
from jax.experimental.pallas import tpu_sc as plsc

N = 10000
E = 320000
NPAD = 10240
NROWS = NPAD // 16
E_PER_TILE = E // 16
CHUNK = 128
NCHUNKS = 160
SUPER = 32
NGRP = SUPER // 8
EPT_PAD = NCHUNKS * CHUNK
DEG_W = 16


def _sc_mesh():
    return plsc.VectorSubcoreMesh(core_axis_name="c", subcore_axis_name="s")


@functools.partial(
    pl.kernel,
    mesh=_sc_mesh(),
    out_type=(
        jax.ShapeDtypeStruct((NPAD, DEG_W), jnp.float32),
        jax.ShapeDtypeStruct((NPAD, DEG_W), jnp.float32),
    ),
    scratch_types=[
        pltpu.VMEM((NCHUNKS, CHUNK), jnp.int32),
        pltpu.VMEM((CHUNK, DEG_W), jnp.float32),
        pltpu.VMEM_SHARED((NPAD, DEG_W), jnp.float32),
    ],
    compiler_params=pltpu.CompilerParams(use_tc_tiling_on_sc=False),
)
def _sc_degrees(src_h, dst_h, ones_h, zero_h, out_src, out_dst, idx_v, ones_v,
                acc):
    c = lax.axis_index("c")
    s = lax.axis_index("s")
    pltpu.sync_copy(zero_h, acc.at[pl.ds(s * NROWS, NROWS)])
    pltpu.sync_copy(ones_h, ones_v)
    plsc.subcore_barrier()

    def run(e_h, out_h):
        pltpu.sync_copy(e_h.at[s], idx_v)

        def body(i, carry):
            pltpu.sync_copy(ones_v, acc.at[idx_v.at[i]], add=True)
            return carry

        lax.fori_loop(0, NCHUNKS, body, 0)
        plsc.subcore_barrier()
        pltpu.sync_copy(acc.at[pl.ds(s * NROWS, NROWS)],
                        out_h.at[pl.ds(s * NROWS, NROWS)])

    @pl.when(c == 0)
    def _():
        run(src_h, out_src)

    @pl.when(c == 1)
    def _():
        run(dst_h, out_dst)


def _make_propagate(f_half, chunk, ring, nchunks, super_):
    nsuper = nchunks // super_
    lag = ring // 2
    ngrp = super_ // ring

    @functools.partial(
        pl.kernel,
        mesh=_sc_mesh(),
        out_type=(
            jax.ShapeDtypeStruct((NPAD, f_half), jnp.float32),
            jax.ShapeDtypeStruct((NPAD, f_half), jnp.float32),
        ),
        scratch_types=[
            pltpu.VMEM((super_, chunk), jnp.int32),
            pltpu.VMEM((super_, chunk), jnp.int32),
            [pltpu.VMEM((chunk, f_half), jnp.float32) for _ in range(ring)],
            [pltpu.SemaphoreType.DMA for _ in range(ring)],
            [pltpu.SemaphoreType.DMA for _ in range(ring)],
            pltpu.VMEM_SHARED((NPAD, f_half), jnp.float32),
        ],
        compiler_params=pltpu.CompilerParams(use_tc_tiling_on_sc=False),
    )
    def prop(ha, hb, srcA, dstA, srcB, dstB, zero_h, outa, outb,
             sidx, didx, rows, gsem, ssem, acc):
        c = lax.axis_index("c")
        s = lax.axis_index("s")
        pltpu.sync_copy(zero_h, acc.at[pl.ds(s * NROWS, NROWS)])
        plsc.subcore_barrier()

        def run(h_h, src_h, dst_h, out_h):
            def gather(i, b):
                pltpu.async_copy(h_h.at[sidx.at[i]], rows[b], gsem[b])

            def wait_gather(i, b):
                pltpu.make_async_copy(h_h.at[sidx.at[i]], rows[b],
                                      gsem[b]).wait()

            def scatter(i, b):
                pltpu.async_copy(rows[b], acc.at[didx.at[i]], ssem[b],
                                 add=True)

            def wait_scatter(i, b):
                pltpu.make_async_copy(rows[b], acc.at[didx.at[i]],
                                      ssem[b]).wait()

            def souter(t, carry):
                pltpu.sync_copy(src_h.at[s, pl.ds(t * super_, super_)], sidx)
                pltpu.sync_copy(dst_h.at[s, pl.ds(t * super_, super_)], didx)
                for b in range(lag):
                    gather(b, b)

                def group(m, carry2):
                    base = ring * m
                    for b in range(lag):
                        i = base + b

                        @pl.when(m > 0)
                        def _(i=i, b=b):
                            wait_scatter(i - lag, b + lag)

                        gather(i + lag, b + lag)
                        wait_gather(i, b)
                        scatter(i, b)
                    for b in range(lag, ring):
                        i = base + b

                        @pl.when(m < ngrp - 1)
                        def _(i=i, b=b):
                            wait_scatter(i - lag, b - lag)
                            gather(i + lag, b - lag)

                        wait_gather(i, b)
                        scatter(i, b)
                    return carry2

                lax.fori_loop(0, ngrp, group, 0)
                for b in range(ring):
                    wait_scatter(super_ - ring + b, b)
                return carry

            lax.fori_loop(0, nsuper, souter, 0)
            plsc.subcore_barrier()
            pltpu.sync_copy(acc.at[pl.ds(s * NROWS, NROWS)],
                            out_h.at[pl.ds(s * NROWS, NROWS)])

        @pl.when(c == 0)
        def _():
            run(ha, srcA, dstA, outa)

        @pl.when(c == 1)
        def _():
            run(hb, srcB, dstB, outb)

    return prop


_prop_l0 = _make_propagate(64, 128, 8, 160, 32)
_prop_l1 = _make_propagate(128, 64, 4, 320, 32)
_prop_l2 = _make_propagate(32, 128, 8, 160, 32)


RB = 1000
GRID = 10


def _row_spec(w):
    return pl.BlockSpec((RB, w), lambda i: (i, 0))


def _full_spec(shape):
    return pl.BlockSpec(shape, lambda i: tuple(0 for _ in shape))


def _tc_prep(x, od_h, id_h):
    def body(x_ref, od_ref, id_ref, ua_ref, ub_ref, ns_ref, nd_ref):
        ns = lax.rsqrt(jnp.maximum(od_ref[...][:, :1], 1.0))
        nd = lax.rsqrt(jnp.maximum(id_ref[...][:, :1], 1.0))
        ns_ref[...] = ns
        nd_ref[...] = nd
        u = x_ref[...] * ns
        ua_ref[...] = u[:, :64]
        ub_ref[...] = u[:, 64:]

    return pl.pallas_call(
        body,
        grid=(GRID,),
        in_specs=[_row_spec(128), _row_spec(DEG_W), _row_spec(DEG_W)],
        out_specs=[_row_spec(64), _row_spec(64), _row_spec(1), _row_spec(1)],
        out_shape=[
            jax.ShapeDtypeStruct((NPAD, 64), jnp.float32),
            jax.ShapeDtypeStruct((NPAD, 64), jnp.float32),
            jax.ShapeDtypeStruct((N, 1), jnp.float32),
            jax.ShapeDtypeStruct((N, 1), jnp.float32),
        ],
    )(x, od_h, id_h)


def _tc_layer0(ya, yb, ns, nd, W0, b0):
    def body(ya_ref, yb_ref, ns_ref, nd_ref, w_ref, b_ref, ua_ref, ub_ref):
        z = jnp.dot(ya_ref[...], w_ref[:64, :], preferred_element_type=jnp.float32)
        z += jnp.dot(yb_ref[...], w_ref[64:, :], preferred_element_type=jnp.float32)
        z = z * nd_ref[...] + b_ref[...]
        u = jnp.maximum(z, 0.0) * ns_ref[...]
        ua_ref[...] = u[:, :128]
        ub_ref[...] = u[:, 128:]

    return pl.pallas_call(
        body,
        grid=(GRID,),
        in_specs=[_row_spec(64), _row_spec(64), _row_spec(1), _row_spec(1),
                  _full_spec((128, 256)), _full_spec((1, 256))],
        out_specs=[_row_spec(128), _row_spec(128)],
        out_shape=[
            jax.ShapeDtypeStruct((NPAD, 128), jnp.float32),
            jax.ShapeDtypeStruct((NPAD, 128), jnp.float32),
        ],
    )(ya, yb, ns, nd, W0, b0)


def _tc_layer12(ya, yb, ns, nd, W1, b1, W2):
    def body(ya_ref, yb_ref, ns_ref, nd_ref, w1_ref, b1_ref, w2_ref,
             ua_ref, ub_ref):
        z = jnp.dot(ya_ref[...], w1_ref[:128, :], preferred_element_type=jnp.float32)
        z += jnp.dot(yb_ref[...], w1_ref[128:, :], preferred_element_type=jnp.float32)
        z = z * nd_ref[...] + b1_ref[...]
        g = jnp.maximum(z, 0.0) * ns_ref[...]
        u = jnp.dot(g, w2_ref[...], preferred_element_type=jnp.float32)
        ua_ref[...] = u[:, :32]
        ub_ref[...] = u[:, 32:]

    return pl.pallas_call(
        body,
        grid=(GRID,),
        in_specs=[_row_spec(128), _row_spec(128), _row_spec(1), _row_spec(1),
                  _full_spec((256, 256)), _full_spec((1, 256)),
                  _full_spec((256, 64))],
        out_specs=[_row_spec(32), _row_spec(32)],
        out_shape=[
            jax.ShapeDtypeStruct((NPAD, 32), jnp.float32),
            jax.ShapeDtypeStruct((NPAD, 32), jnp.float32),
        ],
    )(ya, yb, ns, nd, W1, b1, W2)


def _tc_final(ya, yb, nd, b2):
    def body(ya_ref, yb_ref, nd_ref, b_ref, o_ref):
        z = jnp.concatenate([ya_ref[...], yb_ref[...]], axis=1)
        z = z * nd_ref[...] + b_ref[...]
        m = jnp.max(z, axis=1, keepdims=True)
        e = jnp.exp(z - m)
        lse = jnp.log(jnp.sum(e, axis=1, keepdims=True))
        o_ref[...] = (z - m) - lse

    return pl.pallas_call(
        body,
        grid=(GRID,),
        in_specs=[_row_spec(32), _row_spec(32), _row_spec(1),
                  _full_spec((1, 64))],
        out_specs=_row_spec(64),
        out_shape=jax.ShapeDtypeStruct((N, 64), jnp.float32),
    )(ya, yb, nd, b2)


def kernel(x, edge_index, W0, b0, W1, b1, W2, b2):
    pad = EPT_PAD - E_PER_TILE
    src3 = jnp.pad(edge_index[0].reshape(16, E_PER_TILE), ((0, 0), (0, pad)),
                   constant_values=N).reshape(16, NCHUNKS, CHUNK)
    dst3 = jnp.pad(edge_index[1].reshape(16, E_PER_TILE), ((0, 0), (0, pad)),
                   constant_values=N).reshape(16, NCHUNKS, CHUNK)
    ones_chunk = jnp.zeros((CHUNK, DEG_W), jnp.float32).at[:, 0].set(1.0)
    zero_deg = jnp.zeros((NROWS, DEG_W), jnp.float32)
    od_h, id_h = _sc_degrees(src3, dst3, ones_chunk, zero_deg)

    ua, ub, ns, nd = _tc_prep(x, od_h, id_h)

    y0a, y0b = _prop_l0(ua, ub, src3, dst3, src3, dst3,
                        jnp.zeros((NROWS, 64), jnp.float32))
    u1a, u1b = _tc_layer0(y0a, y0b, ns, nd, W0, b0.reshape(1, -1))

    src3b = src3.reshape(16, 320, 64)
    dst3b = dst3.reshape(16, 320, 64)
    y1a, y1b = _prop_l1(u1a, u1b, src3b, dst3b, src3b, dst3b,
                        jnp.zeros((NROWS, 128), jnp.float32))
    u2a, u2b = _tc_layer12(y1a, y1b, ns, nd, W1, b1.reshape(1, -1), W2)

    y2a, y2b = _prop_l2(u2a, u2b, src3, dst3, src3, dst3,
                        jnp.zeros((NROWS, 32), jnp.float32))
    return _tc_final(y2a, y2b, nd, b2.reshape(1, -1))


# speedup vs baseline: 1.8523x; 1.5504x over previous
import functools

import jax
import jax.numpy as jnp
from jax import lax
from jax.experimental import pallas as pl
from jax.experimental.pallas import tpu as pltpu
from jax.experimental.pallas import tpu_sc as plsc

N = 10000
E = 320000
NPAD = 10240
NROWS = NPAD // 16
E_PER_TILE = E // 16
CHUNK = 128
NCHUNKS = 160
SUPER = 32
NGRP = SUPER // 8
EPT_PAD = NCHUNKS * CHUNK
DEG_W = 16


def _sc_mesh():
    return plsc.VectorSubcoreMesh(core_axis_name="c", subcore_axis_name="s")


@functools.partial(
    pl.kernel,
    mesh=_sc_mesh(),
    out_type=(
        jax.ShapeDtypeStruct((NPAD, DEG_W), jnp.float32),
        jax.ShapeDtypeStruct((NPAD, DEG_W), jnp.float32),
    ),
    scratch_types=[
        pltpu.VMEM((NCHUNKS, CHUNK), jnp.int32),
        pltpu.VMEM((CHUNK, DEG_W), jnp.float32),
        pltpu.VMEM_SHARED((NPAD, DEG_W), jnp.float32),
    ],
    compiler_params=pltpu.CompilerParams(use_tc_tiling_on_sc=False),
)
def _sc_degrees(src_h, dst_h, ones_h, zero_h, out_src, out_dst, idx_v, ones_v,
                acc):
    c = lax.axis_index("c")
    s = lax.axis_index("s")
    pltpu.sync_copy(zero_h, acc.at[pl.ds(s * NROWS, NROWS)])
    pltpu.sync_copy(ones_h, ones_v)
    plsc.subcore_barrier()

    def run(e_h, out_h):
        pltpu.sync_copy(e_h.at[s], idx_v)

        def body(i, carry):
            pltpu.sync_copy(ones_v, acc.at[idx_v.at[i]], add=True)
            return carry

        lax.fori_loop(0, NCHUNKS, body, 0)
        plsc.subcore_barrier()
        pltpu.sync_copy(acc.at[pl.ds(s * NROWS, NROWS)],
                        out_h.at[pl.ds(s * NROWS, NROWS)])

    @pl.when(c == 0)
    def _():
        run(src_h, out_src)

    @pl.when(c == 1)
    def _():
        run(dst_h, out_dst)


def _make_propagate(f_half, chunk, ring, nchunks, super_, stage_table=False):
    nsuper = nchunks // super_
    lag = ring // 2
    ngrp = super_ // ring

    @functools.partial(
        pl.kernel,
        mesh=_sc_mesh(),
        out_type=(
            jax.ShapeDtypeStruct((NPAD, f_half), jnp.float32),
            jax.ShapeDtypeStruct((NPAD, f_half), jnp.float32),
        ),
        scratch_types=[
            pltpu.VMEM((super_, chunk), jnp.int32),
            pltpu.VMEM((super_, chunk), jnp.int32),
            [pltpu.VMEM((chunk, f_half), jnp.float32) for _ in range(ring)],
            [pltpu.SemaphoreType.DMA for _ in range(ring)],
            [pltpu.SemaphoreType.DMA for _ in range(ring)],
            pltpu.VMEM_SHARED((NPAD, f_half), jnp.float32),
        ] + ([pltpu.VMEM_SHARED((NPAD, f_half), jnp.float32)]
             if stage_table else []),
        compiler_params=pltpu.CompilerParams(use_tc_tiling_on_sc=False),
    )
    def prop(ha, hb, srcA, dstA, srcB, dstB, zero_h, outa, outb,
             sidx, didx, rows, gsem, ssem, acc, *maybe_tab):
        c = lax.axis_index("c")
        s = lax.axis_index("s")
        pltpu.sync_copy(zero_h, acc.at[pl.ds(s * NROWS, NROWS)])
        plsc.subcore_barrier()

        def run(h_h, src_h, dst_h, out_h):
            if stage_table:
                tab = maybe_tab[0]
                pltpu.sync_copy(h_h.at[pl.ds(s * NROWS, NROWS)],
                                tab.at[pl.ds(s * NROWS, NROWS)])
                plsc.subcore_barrier()
                table = tab
            else:
                table = h_h

            def gather(i, b):
                pltpu.async_copy(table.at[sidx.at[i]], rows[b], gsem[b])

            def wait_gather(i, b):
                pltpu.make_async_copy(table.at[sidx.at[i]], rows[b],
                                      gsem[b]).wait()

            def scatter(i, b):
                pltpu.async_copy(rows[b], acc.at[didx.at[i]], ssem[b],
                                 add=True)

            def wait_scatter(i, b):
                pltpu.make_async_copy(rows[b], acc.at[didx.at[i]],
                                      ssem[b]).wait()

            def souter(t, carry):
                pltpu.sync_copy(src_h.at[s, pl.ds(t * super_, super_)], sidx)
                pltpu.sync_copy(dst_h.at[s, pl.ds(t * super_, super_)], didx)
                for b in range(lag):
                    gather(b, b)

                def group(m, carry2):
                    base = ring * m
                    for b in range(lag):
                        i = base + b

                        @pl.when(m > 0)
                        def _(i=i, b=b):
                            wait_scatter(i - lag, b + lag)

                        gather(i + lag, b + lag)
                        wait_gather(i, b)
                        scatter(i, b)
                    for b in range(lag, ring):
                        i = base + b

                        @pl.when(m < ngrp - 1)
                        def _(i=i, b=b):
                            wait_scatter(i - lag, b - lag)
                            gather(i + lag, b - lag)

                        wait_gather(i, b)
                        scatter(i, b)
                    return carry2

                lax.fori_loop(0, ngrp, group, 0)
                for b in range(ring):
                    wait_scatter(super_ - ring + b, b)
                return carry

            lax.fori_loop(0, nsuper, souter, 0)
            plsc.subcore_barrier()
            pltpu.sync_copy(acc.at[pl.ds(s * NROWS, NROWS)],
                            out_h.at[pl.ds(s * NROWS, NROWS)])

        @pl.when(c == 0)
        def _():
            run(ha, srcA, dstA, outa)

        @pl.when(c == 1)
        def _():
            run(hb, srcB, dstB, outb)

    return prop


_prop64q = _make_propagate(64, 128, 4, 160, 32, stage_table=True)
_prop_l2 = _make_propagate(32, 128, 8, 160, 32, stage_table=True)


RB = 1000
GRID = 10


def _row_spec(w):
    return pl.BlockSpec((RB, w), lambda i: (i, 0))


def _full_spec(shape):
    return pl.BlockSpec(shape, lambda i: tuple(0 for _ in shape))


def _tc_prep(x, od_h, id_h):
    def body(x_ref, od_ref, id_ref, ua_ref, ub_ref, ns_ref, nd_ref):
        ns = lax.rsqrt(jnp.maximum(od_ref[...][:, :1], 1.0))
        nd = lax.rsqrt(jnp.maximum(id_ref[...][:, :1], 1.0))
        ns_ref[...] = ns
        nd_ref[...] = nd
        u = x_ref[...] * ns
        ua_ref[...] = u[:, :64]
        ub_ref[...] = u[:, 64:]

    return pl.pallas_call(
        body,
        grid=(GRID,),
        in_specs=[_row_spec(128), _row_spec(DEG_W), _row_spec(DEG_W)],
        out_specs=[_row_spec(64), _row_spec(64), _row_spec(1), _row_spec(1)],
        out_shape=[
            jax.ShapeDtypeStruct((NPAD, 64), jnp.float32),
            jax.ShapeDtypeStruct((NPAD, 64), jnp.float32),
            jax.ShapeDtypeStruct((N, 1), jnp.float32),
            jax.ShapeDtypeStruct((N, 1), jnp.float32),
        ],
    )(x, od_h, id_h)


def _tc_layer0(ya, yb, ns, nd, W0, b0):
    def body(ya_ref, yb_ref, ns_ref, nd_ref, w_ref, b_ref,
             u0_ref, u1_ref, u2_ref, u3_ref):
        z = jnp.dot(ya_ref[...], w_ref[:64, :], preferred_element_type=jnp.float32)
        z += jnp.dot(yb_ref[...], w_ref[64:, :], preferred_element_type=jnp.float32)
        z = z * nd_ref[...] + b_ref[...]
        u = jnp.maximum(z, 0.0) * ns_ref[...]
        u0_ref[...] = u[:, 0:64]
        u1_ref[...] = u[:, 64:128]
        u2_ref[...] = u[:, 128:192]
        u3_ref[...] = u[:, 192:256]

    return pl.pallas_call(
        body,
        grid=(GRID,),
        in_specs=[_row_spec(64), _row_spec(64), _row_spec(1), _row_spec(1),
                  _full_spec((128, 256)), _full_spec((1, 256))],
        out_specs=[_row_spec(64)] * 4,
        out_shape=[jax.ShapeDtypeStruct((NPAD, 64), jnp.float32)] * 4,
    )(ya, yb, ns, nd, W0, b0)


def _tc_layer12(y0, y1, y2, y3, ns, nd, W1, b1, W2):
    def body(y0_ref, y1_ref, y2_ref, y3_ref, ns_ref, nd_ref,
             w1_ref, b1_ref, w2_ref, ua_ref, ub_ref):
        z = jnp.dot(y0_ref[...], w1_ref[0:64, :], preferred_element_type=jnp.float32)
        z += jnp.dot(y1_ref[...], w1_ref[64:128, :], preferred_element_type=jnp.float32)
        z += jnp.dot(y2_ref[...], w1_ref[128:192, :], preferred_element_type=jnp.float32)
        z += jnp.dot(y3_ref[...], w1_ref[192:256, :], preferred_element_type=jnp.float32)
        z = z * nd_ref[...] + b1_ref[...]
        g = jnp.maximum(z, 0.0) * ns_ref[...]
        u = jnp.dot(g, w2_ref[...], preferred_element_type=jnp.float32)
        ua_ref[...] = u[:, :32]
        ub_ref[...] = u[:, 32:]

    return pl.pallas_call(
        body,
        grid=(GRID,),
        in_specs=[_row_spec(64)] * 4 + [_row_spec(1), _row_spec(1),
                  _full_spec((256, 256)), _full_spec((1, 256)),
                  _full_spec((256, 64))],
        out_specs=[_row_spec(32), _row_spec(32)],
        out_shape=[
            jax.ShapeDtypeStruct((NPAD, 32), jnp.float32),
            jax.ShapeDtypeStruct((NPAD, 32), jnp.float32),
        ],
    )(y0, y1, y2, y3, ns, nd, W1, b1, W2)


def _tc_final(ya, yb, nd, b2):
    def body(ya_ref, yb_ref, nd_ref, b_ref, o_ref):
        z = jnp.concatenate([ya_ref[...], yb_ref[...]], axis=1)
        z = z * nd_ref[...] + b_ref[...]
        m = jnp.max(z, axis=1, keepdims=True)
        e = jnp.exp(z - m)
        lse = jnp.log(jnp.sum(e, axis=1, keepdims=True))
        o_ref[...] = (z - m) - lse

    return pl.pallas_call(
        body,
        grid=(GRID,),
        in_specs=[_row_spec(32), _row_spec(32), _row_spec(1),
                  _full_spec((1, 64))],
        out_specs=_row_spec(64),
        out_shape=jax.ShapeDtypeStruct((N, 64), jnp.float32),
    )(ya, yb, nd, b2)


def kernel(x, edge_index, W0, b0, W1, b1, W2, b2):
    pad = EPT_PAD - E_PER_TILE
    src3 = jnp.pad(edge_index[0].reshape(16, E_PER_TILE), ((0, 0), (0, pad)),
                   constant_values=N).reshape(16, NCHUNKS, CHUNK)
    dst3 = jnp.pad(edge_index[1].reshape(16, E_PER_TILE), ((0, 0), (0, pad)),
                   constant_values=N).reshape(16, NCHUNKS, CHUNK)
    ones_chunk = jnp.zeros((CHUNK, DEG_W), jnp.float32).at[:, 0].set(1.0)
    zero_deg = jnp.zeros((NROWS, DEG_W), jnp.float32)
    od_h, id_h = _sc_degrees(src3, dst3, ones_chunk, zero_deg)

    ua, ub, ns, nd = _tc_prep(x, od_h, id_h)

    zero64 = jnp.zeros((NROWS, 64), jnp.float32)
    y0a, y0b = _prop64q(ua, ub, src3, dst3, src3, dst3, zero64)
    q0, q1, q2, q3 = _tc_layer0(y0a, y0b, ns, nd, W0, b0.reshape(1, -1))

    y1q0, y1q1 = _prop64q(q0, q1, src3, dst3, src3, dst3, zero64)
    y1q2, y1q3 = _prop64q(q2, q3, src3, dst3, src3, dst3, zero64)
    u2a, u2b = _tc_layer12(y1q0, y1q1, y1q2, y1q3, ns, nd,
                           W1, b1.reshape(1, -1), W2)

    y2a, y2b = _prop_l2(u2a, u2b, src3, dst3, src3, dst3,
                        jnp.zeros((NROWS, 32), jnp.float32))
    return _tc_final(y2a, y2b, nd, b2.reshape(1, -1))


# speedup vs baseline: 1.8644x; 1.0065x over previous
import functools

import jax
import jax.numpy as jnp
from jax import lax
from jax.experimental import pallas as pl
from jax.experimental.pallas import tpu as pltpu
from jax.experimental.pallas import tpu_sc as plsc

N = 10000
E = 320000
NPAD = 10240
NROWS = NPAD // 16
E_PER_TILE = E // 16
CHUNK = 128
NCHUNKS = 160
SUPER = 32
NGRP = SUPER // 8
EPT_PAD = NCHUNKS * CHUNK
DEG_W = 16


def _sc_mesh():
    return plsc.VectorSubcoreMesh(core_axis_name="c", subcore_axis_name="s")


@functools.partial(
    pl.kernel,
    mesh=_sc_mesh(),
    out_type=(
        jax.ShapeDtypeStruct((NPAD, DEG_W), jnp.float32),
        jax.ShapeDtypeStruct((NPAD, DEG_W), jnp.float32),
    ),
    scratch_types=[
        pltpu.VMEM((NCHUNKS, CHUNK), jnp.int32),
        pltpu.VMEM((CHUNK, DEG_W), jnp.float32),
        pltpu.VMEM_SHARED((NPAD, DEG_W), jnp.float32),
    ],
    compiler_params=pltpu.CompilerParams(use_tc_tiling_on_sc=False),
)
def _sc_degrees(src_h, dst_h, ones_h, zero_h, out_src, out_dst, idx_v, ones_v,
                acc):
    c = lax.axis_index("c")
    s = lax.axis_index("s")
    pltpu.sync_copy(zero_h, acc.at[pl.ds(s * NROWS, NROWS)])
    pltpu.sync_copy(ones_h, ones_v)
    plsc.subcore_barrier()

    def run(e_h, out_h):
        pltpu.sync_copy(e_h.at[s], idx_v)

        def body(i, carry):
            pltpu.sync_copy(ones_v, acc.at[idx_v.at[i]], add=True)
            return carry

        lax.fori_loop(0, NCHUNKS, body, 0)
        plsc.subcore_barrier()
        pltpu.sync_copy(acc.at[pl.ds(s * NROWS, NROWS)],
                        out_h.at[pl.ds(s * NROWS, NROWS)])

    @pl.when(c == 0)
    def _():
        run(src_h, out_src)

    @pl.when(c == 1)
    def _():
        run(dst_h, out_dst)


def _make_propagate(f_half, chunk, ring, nchunks, super_, stage_table=False,
                    npass=1):
    nsuper = nchunks // super_
    lag = ring // 2
    ngrp = super_ // ring

    @functools.partial(
        pl.kernel,
        mesh=_sc_mesh(),
        out_type=tuple(jax.ShapeDtypeStruct((NPAD, f_half), jnp.float32)
                       for _ in range(2 * npass)),
        scratch_types=[
            pltpu.VMEM((super_, chunk), jnp.int32),
            pltpu.VMEM((super_, chunk), jnp.int32),
            [pltpu.VMEM((chunk, f_half), jnp.float32) for _ in range(ring)],
            [pltpu.SemaphoreType.DMA for _ in range(ring)],
            [pltpu.SemaphoreType.DMA for _ in range(ring)],
            pltpu.VMEM_SHARED((NPAD, f_half), jnp.float32),
        ] + ([pltpu.VMEM_SHARED((NPAD, f_half), jnp.float32)]
             if stage_table else []),
        compiler_params=pltpu.CompilerParams(use_tc_tiling_on_sc=False),
    )
    def prop(*args):
        tables = args[:2 * npass]
        srcA, dstA, srcB, dstB, zero_h = args[2 * npass:2 * npass + 5]
        outs = args[2 * npass + 5:4 * npass + 5]
        sidx, didx, rows, gsem, ssem, acc = args[4 * npass + 5:4 * npass + 11]
        maybe_tab = args[4 * npass + 11:]
        c = lax.axis_index("c")
        s = lax.axis_index("s")

        def run(h_h, src_h, dst_h, out_h):
            pltpu.sync_copy(zero_h, acc.at[pl.ds(s * NROWS, NROWS)])
            if stage_table:
                tab = maybe_tab[0]
                pltpu.sync_copy(h_h.at[pl.ds(s * NROWS, NROWS)],
                                tab.at[pl.ds(s * NROWS, NROWS)])
                table = tab
            else:
                table = h_h
            plsc.subcore_barrier()

            def gather(i, b):
                pltpu.async_copy(table.at[sidx.at[i]], rows[b], gsem[b])

            def wait_gather(i, b):
                pltpu.make_async_copy(table.at[sidx.at[i]], rows[b],
                                      gsem[b]).wait()

            def scatter(i, b):
                pltpu.async_copy(rows[b], acc.at[didx.at[i]], ssem[b],
                                 add=True)

            def wait_scatter(i, b):
                pltpu.make_async_copy(rows[b], acc.at[didx.at[i]],
                                      ssem[b]).wait()

            def souter(t, carry):
                pltpu.sync_copy(src_h.at[s, pl.ds(t * super_, super_)], sidx)
                pltpu.sync_copy(dst_h.at[s, pl.ds(t * super_, super_)], didx)
                for b in range(lag):
                    gather(b, b)

                def group(m, carry2):
                    base = ring * m
                    for b in range(lag):
                        i = base + b

                        @pl.when(m > 0)
                        def _(i=i, b=b):
                            wait_scatter(i - lag, b + lag)

                        gather(i + lag, b + lag)
                        wait_gather(i, b)
                        scatter(i, b)
                    for b in range(lag, ring):
                        i = base + b

                        @pl.when(m < ngrp - 1)
                        def _(i=i, b=b):
                            wait_scatter(i - lag, b - lag)
                            gather(i + lag, b - lag)

                        wait_gather(i, b)
                        scatter(i, b)
                    return carry2

                lax.fori_loop(0, ngrp, group, 0)
                for b in range(ring):
                    wait_scatter(super_ - ring + b, b)
                return carry

            lax.fori_loop(0, nsuper, souter, 0)
            plsc.subcore_barrier()
            pltpu.sync_copy(acc.at[pl.ds(s * NROWS, NROWS)],
                            out_h.at[pl.ds(s * NROWS, NROWS)])

        @pl.when(c == 0)
        def _():
            for p in range(npass):
                run(tables[2 * p], srcA, dstA, outs[2 * p])

        @pl.when(c == 1)
        def _():
            for p in range(npass):
                run(tables[2 * p + 1], srcB, dstB, outs[2 * p + 1])

    return prop


_prop64q = _make_propagate(64, 128, 4, 160, 32, stage_table=True)
_prop64q2 = _make_propagate(64, 128, 4, 160, 32, stage_table=True, npass=2)
_prop_l2 = _make_propagate(32, 128, 8, 160, 32, stage_table=True)


RB = 1000
GRID = 10


def _row_spec(w):
    return pl.BlockSpec((RB, w), lambda i: (i, 0))


def _full_spec(shape):
    return pl.BlockSpec(shape, lambda i: tuple(0 for _ in shape))


def _tc_prep(x, od_h, id_h):
    def body(x_ref, od_ref, id_ref, ua_ref, ub_ref, ns_ref, nd_ref):
        ns = lax.rsqrt(jnp.maximum(od_ref[...][:, :1], 1.0))
        nd = lax.rsqrt(jnp.maximum(id_ref[...][:, :1], 1.0))
        ns_ref[...] = ns
        nd_ref[...] = nd
        u = x_ref[...] * ns
        ua_ref[...] = u[:, :64]
        ub_ref[...] = u[:, 64:]

    return pl.pallas_call(
        body,
        grid=(GRID,),
        in_specs=[_row_spec(128), _row_spec(DEG_W), _row_spec(DEG_W)],
        out_specs=[_row_spec(64), _row_spec(64), _row_spec(1), _row_spec(1)],
        out_shape=[
            jax.ShapeDtypeStruct((NPAD, 64), jnp.float32),
            jax.ShapeDtypeStruct((NPAD, 64), jnp.float32),
            jax.ShapeDtypeStruct((N, 1), jnp.float32),
            jax.ShapeDtypeStruct((N, 1), jnp.float32),
        ],
    )(x, od_h, id_h)


def _tc_layer0(ya, yb, ns, nd, W0, b0):
    def body(ya_ref, yb_ref, ns_ref, nd_ref, w_ref, b_ref,
             u0_ref, u1_ref, u2_ref, u3_ref):
        z = jnp.dot(ya_ref[...], w_ref[:64, :], preferred_element_type=jnp.float32)
        z += jnp.dot(yb_ref[...], w_ref[64:, :], preferred_element_type=jnp.float32)
        z = z * nd_ref[...] + b_ref[...]
        u = jnp.maximum(z, 0.0) * ns_ref[...]
        u0_ref[...] = u[:, 0:64]
        u1_ref[...] = u[:, 64:128]
        u2_ref[...] = u[:, 128:192]
        u3_ref[...] = u[:, 192:256]

    return pl.pallas_call(
        body,
        grid=(GRID,),
        in_specs=[_row_spec(64), _row_spec(64), _row_spec(1), _row_spec(1),
                  _full_spec((128, 256)), _full_spec((1, 256))],
        out_specs=[_row_spec(64)] * 4,
        out_shape=[jax.ShapeDtypeStruct((NPAD, 64), jnp.float32)] * 4,
    )(ya, yb, ns, nd, W0, b0)


def _tc_layer12(y0, y1, y2, y3, ns, nd, W1, b1, W2):
    def body(y0_ref, y1_ref, y2_ref, y3_ref, ns_ref, nd_ref,
             w1_ref, b1_ref, w2_ref, ua_ref, ub_ref):
        z = jnp.dot(y0_ref[...], w1_ref[0:64, :], preferred_element_type=jnp.float32)
        z += jnp.dot(y1_ref[...], w1_ref[64:128, :], preferred_element_type=jnp.float32)
        z += jnp.dot(y2_ref[...], w1_ref[128:192, :], preferred_element_type=jnp.float32)
        z += jnp.dot(y3_ref[...], w1_ref[192:256, :], preferred_element_type=jnp.float32)
        z = z * nd_ref[...] + b1_ref[...]
        g = jnp.maximum(z, 0.0) * ns_ref[...]
        u = jnp.dot(g, w2_ref[...], preferred_element_type=jnp.float32)
        ua_ref[...] = u[:, :32]
        ub_ref[...] = u[:, 32:]

    return pl.pallas_call(
        body,
        grid=(GRID,),
        in_specs=[_row_spec(64)] * 4 + [_row_spec(1), _row_spec(1),
                  _full_spec((256, 256)), _full_spec((1, 256)),
                  _full_spec((256, 64))],
        out_specs=[_row_spec(32), _row_spec(32)],
        out_shape=[
            jax.ShapeDtypeStruct((NPAD, 32), jnp.float32),
            jax.ShapeDtypeStruct((NPAD, 32), jnp.float32),
        ],
    )(y0, y1, y2, y3, ns, nd, W1, b1, W2)


def _tc_final(ya, yb, nd, b2):
    def body(ya_ref, yb_ref, nd_ref, b_ref, o_ref):
        z = jnp.concatenate([ya_ref[...], yb_ref[...]], axis=1)
        z = z * nd_ref[...] + b_ref[...]
        m = jnp.max(z, axis=1, keepdims=True)
        e = jnp.exp(z - m)
        lse = jnp.log(jnp.sum(e, axis=1, keepdims=True))
        o_ref[...] = (z - m) - lse

    return pl.pallas_call(
        body,
        grid=(GRID,),
        in_specs=[_row_spec(32), _row_spec(32), _row_spec(1),
                  _full_spec((1, 64))],
        out_specs=_row_spec(64),
        out_shape=jax.ShapeDtypeStruct((N, 64), jnp.float32),
    )(ya, yb, nd, b2)


def kernel(x, edge_index, W0, b0, W1, b1, W2, b2):
    pad = EPT_PAD - E_PER_TILE
    src3 = jnp.pad(edge_index[0].reshape(16, E_PER_TILE), ((0, 0), (0, pad)),
                   constant_values=N).reshape(16, NCHUNKS, CHUNK)
    dst3 = jnp.pad(edge_index[1].reshape(16, E_PER_TILE), ((0, 0), (0, pad)),
                   constant_values=N).reshape(16, NCHUNKS, CHUNK)
    ones_chunk = jnp.zeros((CHUNK, DEG_W), jnp.float32).at[:, 0].set(1.0)
    zero_deg = jnp.zeros((NROWS, DEG_W), jnp.float32)
    od_h, id_h = _sc_degrees(src3, dst3, ones_chunk, zero_deg)

    ua, ub, ns, nd = _tc_prep(x, od_h, id_h)

    zero64 = jnp.zeros((NROWS, 64), jnp.float32)
    y0a, y0b = _prop64q(ua, ub, src3, dst3, src3, dst3, zero64)
    q0, q1, q2, q3 = _tc_layer0(y0a, y0b, ns, nd, W0, b0.reshape(1, -1))

    y1q0, y1q1, y1q2, y1q3 = _prop64q2(q0, q1, q2, q3,
                                       src3, dst3, src3, dst3, zero64)
    u2a, u2b = _tc_layer12(y1q0, y1q1, y1q2, y1q3, ns, nd,
                           W1, b1.reshape(1, -1), W2)

    y2a, y2b = _prop_l2(u2a, u2b, src3, dst3, src3, dst3,
                        jnp.zeros((NROWS, 32), jnp.float32))
    return _tc_final(y2a, y2b, nd, b2.reshape(1, -1))
